# baseline (speedup 1.0000x reference)
import jax
import jax.numpy as jnp
from jax import lax
from jax.experimental import pallas as pl
from jax.experimental.pallas import tpu as pltpu
from jax.experimental.pallas import tpu_sc as plsc

B, H, W, K, PAD, SP = 2, 384, 512, 7, 3, 768
TEMP, EPS = 10.0, 1e-5

NBAND = 16
BROWS = H // NBAND
NROWS = BROWS + 12
VST_MAX = H - NROWS
NGRP = SP // 16
CAP = SP + 16
MRCHUNK = 112
ACC_N = BROWS * W


def _sc_body(depth_hbm, rc_hbm, mrc_hbm, r_hbm, c_hbm, w_hbm, zero_hbm,
             out_hbm,
             buf_r, buf_c, buf_d, acc_num, acc_cnt, acc_dep,
             rs_v, cs_v, wn_v, gi0, gi1, mr_v, mc_v, e_buf, samp_r,
             samp_c, samp_w, sem_s, sem_b, sem_z, sem_g, sem_o):
    ci = lax.axis_index("c")
    si = lax.axis_index("s")
    b = ci
    lo = si * BROWS
    vstart = jnp.clip(lo - 6, 0, VST_MAX)

    zf = jnp.zeros((16,), jnp.float32)
    onef = jnp.ones((16,), jnp.float32)
    lane = lax.iota(jnp.int32, 16)
    vstartv = jnp.broadcast_to(vstart, (16,))
    lov = jnp.broadcast_to(lo, (16,))

    dbuf = [
        pltpu.async_copy(
            rc_hbm.at[pl.ds((b * 2 * H + vstart) * W, NROWS * W)],
            buf_r, sem_b),
        pltpu.async_copy(
            rc_hbm.at[pl.ds(((b * 2 + 1) * H + vstart) * W, NROWS * W)],
            buf_c, sem_b),
        pltpu.async_copy(
            depth_hbm.at[pl.ds((b * H + vstart) * W, NROWS * W)],
            buf_d, sem_b),
    ]
    dzero = [
        pltpu.async_copy(zero_hbm, acc_num, sem_z),
        pltpu.async_copy(zero_hbm, acc_cnt, sem_z),
        pltpu.async_copy(zero_hbm, acc_dep, sem_z),
    ]
    dsamp = [
        pltpu.async_copy(r_hbm.at[pl.ds(b * SP, SP)], samp_r, sem_s),
        pltpu.async_copy(c_hbm.at[pl.ds(b * SP, SP)], samp_c, sem_s),
        pltpu.async_copy(w_hbm.at[pl.ds(b * SP, SP)], samp_w, sem_s),
    ]

    sent_r = jnp.broadcast_to(vstart, (16,))
    zi = jnp.zeros((16,), jnp.int32)

    def _fill(k, _):
        rs_v[pl.ds(k * 16, 16)] = sent_r
        cs_v[pl.ds(k * 16, 16)] = zi
        wn_v[pl.ds(k * 16, 16)] = zf
        return 0
    lax.fori_loop(0, CAP // 16, _fill, 0)

    for d in dsamp:
        d.wait()

    def _compact(g, cnt):
        off = pl.ds(g * 16, 16)
        rv = samp_r[off]
        cv = samp_c[off]
        wv = samp_w[off]
        sel = (rv >= lov - 3) & (rv < lov + (BROWS + 3))
        pos = plsc.cumsum(sel.astype(jnp.int32))
        dest = cnt + pos - 1
        plsc.store_scatter(rs_v, [dest], rv, mask=sel)
        plsc.store_scatter(cs_v, [dest], cv, mask=sel)
        plsc.store_scatter(wn_v, [dest], wv, mask=sel)
        return cnt + plsc.all_reduce_population_count(sel)
    cnt = lax.fori_loop(0, NGRP, _compact, jnp.zeros((16,), jnp.int32))
    ngroups = ((cnt + 15) // 16)[0]

    basev = jnp.broadcast_to(b * (2 * H * W), (16,))

    def _gidx(k, _):
        off = pl.ds(k * 16, 16)
        fi = rs_v[off] * W + cs_v[off]
        gi0[off] = basev + fi
        gi1[off] = basev + (H * W) + fi
        return 0
    lax.fori_loop(0, CAP // 16, _gidx, 0)
    ch0 = pl.ds(0, MRCHUNK)
    dg = [pltpu.async_copy(mrc_hbm.at[gi0.at[ch0]], mr_v.at[ch0], sem_g),
          pltpu.async_copy(mrc_hbm.at[gi1.at[ch0]], mc_v.at[ch0], sem_g)]
    for i in range(1, CAP // MRCHUNK):
        ch = pl.ds(i * MRCHUNK, MRCHUNK)

        def _fetch(ch=ch):
            pltpu.sync_copy(mrc_hbm.at[gi0.at[ch]], mr_v.at[ch])
            pltpu.sync_copy(mrc_hbm.at[gi1.at[ch]], mc_v.at[ch])
        pl.when(ngroups > i * 7)(_fetch)
    for d in dbuf:
        d.wait()
    for d in dzero:
        d.wait()
    for d in dg:
        d.wait()

    def _group(g, _):
        off = pl.ds(g * 16, 16)
        rv = rs_v[off]
        cv = cs_v[off]
        wv = wn_v[off]
        mr = mr_v[off]
        mc = mc_v[off]
        gvalid = (lane + jnp.broadcast_to(g * 16, (16,))) < cnt

        rbase, rowok, ybandv, ybok = [], [], [], []
        cws, colok = [], []
        for i in range(K):
            wr = rv + (i - PAD)
            rowok.append((wr >= 0) & (wr < H))
            ri = jnp.clip(wr - vstartv, 0, NROWS - 1)
            rbase.append(ri * W)
            yb = wr - lov
            ybok.append((yb >= 0) & (yb < BROWS))
            ybandv.append(yb * W)
        for j in range(K):
            wc = cv + (j - PAD)
            colok.append(wc >= 0)
            cws.append(jnp.maximum(wc, 0))

        esum = zf
        accd = zf
        for k in range(K * K):
            i, j = divmod(k, K)
            gidx = rbase[i] + cws[j]
            ok = rowok[i] & colok[j]
            a = plsc.load_gather(buf_r, [gidx])
            bb = plsc.load_gather(buf_c, [gidx])
            d = plsc.load_gather(buf_d, [gidx])
            a = jnp.where(ok, a, 0.0)
            bb = jnp.where(ok, bb, 0.0)
            d = jnp.where(ok, d, 0.0)
            dr = a - mr
            dc = bb - mc
            e = jnp.exp((dr * dr + dc * dc) * (-1.0 / (TEMP * TEMP)))
            esum = esum + e
            accd = accd + e * d
            e_buf[pl.ds(k * 16, 16)] = e

        inv = 1.0 / (esum + EPS)
        dsoft = accd * inv

        ybc = rv - lov
        cm = gvalid & (ybc >= 0) & (ybc < BROWS)
        plsc.addupdate_scatter(acc_dep, [ybc * W + cv], dsoft, mask=cm)

        wm = gvalid & (wv > 0.5)
        for k in range(K * K):
            i, j = divmod(k, K)
            m = wm & ybok[i] & colok[j]
            sidx = ybandv[i] + jnp.maximum(cv + (j - PAD), 0)
            en = e_buf[pl.ds(k * 16, 16)] * inv
            plsc.addupdate_scatter(acc_num, [sidx], en, mask=m)
            plsc.addupdate_scatter(acc_cnt, [sidx], onef, mask=m)
        return 0
    lax.fori_loop(0, ngroups, _group, 0)

    def _div(i, _):
        for u in range(8):
            off = pl.ds(i * 128 + u * 16, 16)
            acc_num[off] = acc_num[off] / (acc_cnt[off] + EPS)
        return 0
    lax.fori_loop(0, ACC_N // 128, _div, 0)

    do0 = pltpu.async_copy(
        acc_num, out_hbm.at[pl.ds(((b * 2) * H + lo) * W, ACC_N)], sem_o)
    do1 = pltpu.async_copy(
        acc_dep, out_hbm.at[pl.ds(((b * 2 + 1) * H + lo) * W, ACC_N)],
        sem_o)
    do0.wait()
    do1.wait()


_sc_call = pl.kernel(
    _sc_body,
    out_type=jax.ShapeDtypeStruct((B * 2 * H * W,), jnp.float32),
    mesh=plsc.VectorSubcoreMesh(core_axis_name="c", subcore_axis_name="s"),
    compiler_params=pltpu.CompilerParams(needs_layout_passes=False),
    scratch_types=[
        pltpu.VMEM((NROWS * W,), jnp.float32),
        pltpu.VMEM((NROWS * W,), jnp.float32),
        pltpu.VMEM((NROWS * W,), jnp.float32),
        pltpu.VMEM((ACC_N,), jnp.float32),
        pltpu.VMEM((ACC_N,), jnp.float32),
        pltpu.VMEM((ACC_N,), jnp.float32),
        pltpu.VMEM((CAP,), jnp.int32),
        pltpu.VMEM((CAP,), jnp.int32),
        pltpu.VMEM((CAP,), jnp.float32),
        pltpu.VMEM((CAP,), jnp.int32),
        pltpu.VMEM((CAP,), jnp.int32),
        pltpu.VMEM((CAP,), jnp.float32),
        pltpu.VMEM((CAP,), jnp.float32),
        pltpu.VMEM((K * K * 16,), jnp.float32),
        pltpu.VMEM((SP,), jnp.int32),
        pltpu.VMEM((SP,), jnp.int32),
        pltpu.VMEM((SP,), jnp.float32),
        pltpu.SemaphoreType.DMA,
        pltpu.SemaphoreType.DMA,
        pltpu.SemaphoreType.DMA,
        pltpu.SemaphoreType.DMA,
        pltpu.SemaphoreType.DMA,
    ],
)


@jax.jit
def kernel(image_depth, rc_tensor, mask_binary, mask_rc_filled,
           batch_row_col_list):
    del mask_binary
    r = batch_row_col_list[:, 0, :].astype(jnp.int32)
    c = batch_row_col_list[:, 1, :].astype(jnp.int32)
    flat = r * W + c
    s = jnp.arange(SP, dtype=jnp.int32)
    dup_later = ((flat[:, :, None] == flat[:, None, :])
                 & (s[None, :, None] < s[None, None, :]))
    winner = (~jnp.any(dup_later, axis=-1)).astype(jnp.float32)
    out = _sc_call(
        image_depth.reshape(B * H * W),
        rc_tensor.reshape(B * 2 * H * W),
        mask_rc_filled.reshape(B * 2 * H * W),
        r.reshape(B * SP), c.reshape(B * SP), winner.reshape(B * SP),
        jnp.zeros((ACC_N,), jnp.float32),
    )
    return out.reshape(B, 2, H, W)


# speedup vs baseline: 1.1491x; 1.1491x over previous
import jax
import jax.numpy as jnp
from jax import lax
from jax.experimental import pallas as pl
from jax.experimental.pallas import tpu as pltpu
from jax.experimental.pallas import tpu_sc as plsc

B, H, W, K, PAD, SP = 2, 384, 512, 7, 3, 768
TEMP, EPS = 10.0, 1e-5

NBAND = 16
BROWS = H // NBAND
NROWS = BROWS + 12
VST_MAX = H - NROWS
NGRP = SP // 16
CAP = SP + 16
MRCHUNK = 112
ACC_N = BROWS * W


def _sc_body(depth_hbm, rc_hbm, mrc_hbm, r_hbm, c_hbm, w_hbm, out_hbm,
             buf_r, buf_c, buf_d, acc_num, acc_cnt, acc_dep,
             rs_v, cs_v, wn_v, gi0, gi1, mr_v, mc_v, e_buf, samp_r,
             samp_c, samp_w, sem_s, sem_b, sem_g, sem_o):
    ci = lax.axis_index("c")
    si = lax.axis_index("s")
    b = ci
    lo = si * BROWS
    vstart = jnp.clip(lo - 6, 0, VST_MAX)

    zf = jnp.zeros((16,), jnp.float32)
    onef = jnp.ones((16,), jnp.float32)
    lane = lax.iota(jnp.int32, 16)
    vstartv = jnp.broadcast_to(vstart, (16,))
    lov = jnp.broadcast_to(lo, (16,))

    dbuf = [
        pltpu.async_copy(
            rc_hbm.at[pl.ds((b * 2 * H + vstart) * W, NROWS * W)],
            buf_r, sem_b),
        pltpu.async_copy(
            rc_hbm.at[pl.ds(((b * 2 + 1) * H + vstart) * W, NROWS * W)],
            buf_c, sem_b),
        pltpu.async_copy(
            depth_hbm.at[pl.ds((b * H + vstart) * W, NROWS * W)],
            buf_d, sem_b),
    ]
    dsamp = [
        pltpu.async_copy(r_hbm.at[pl.ds(b * SP, SP)], samp_r, sem_s),
        pltpu.async_copy(c_hbm.at[pl.ds(b * SP, SP)], samp_c, sem_s),
        pltpu.async_copy(w_hbm.at[pl.ds(b * SP, SP)], samp_w, sem_s),
    ]

    def _acc_zero(i, _):
        for u in range(8):
            off = pl.ds(i * 128 + u * 16, 16)
            acc_num[off] = zf
            acc_cnt[off] = zf
            acc_dep[off] = zf
        return 0
    lax.fori_loop(0, ACC_N // 128, _acc_zero, 0)

    sent_r = jnp.broadcast_to(vstart, (16,))
    zi = jnp.zeros((16,), jnp.int32)

    def _fill(k, _):
        rs_v[pl.ds(k * 16, 16)] = sent_r
        cs_v[pl.ds(k * 16, 16)] = zi
        wn_v[pl.ds(k * 16, 16)] = zf
        return 0
    lax.fori_loop(0, CAP // 16, _fill, 0)

    for d in dsamp:
        d.wait()

    def _compact(g, cnt):
        off = pl.ds(g * 16, 16)
        rv = samp_r[off]
        cv = samp_c[off]
        wv = samp_w[off]
        sel = (rv >= lov - 3) & (rv < lov + (BROWS + 3))
        pos = plsc.cumsum(sel.astype(jnp.int32))
        dest = cnt + pos - 1
        plsc.store_scatter(rs_v, [dest], rv, mask=sel)
        plsc.store_scatter(cs_v, [dest], cv, mask=sel)
        plsc.store_scatter(wn_v, [dest], wv, mask=sel)
        return cnt + plsc.all_reduce_population_count(sel)
    cnt = lax.fori_loop(0, NGRP, _compact, jnp.zeros((16,), jnp.int32))
    ngroups = ((cnt + 15) // 16)[0]

    basev = jnp.broadcast_to(b * (2 * H * W), (16,))

    def _gidx(k, _):
        off = pl.ds(k * 16, 16)
        fi = rs_v[off] * W + cs_v[off]
        gi0[off] = basev + fi
        gi1[off] = basev + (H * W) + fi
        return 0
    lax.fori_loop(0, CAP // 16, _gidx, 0)
    ch0 = pl.ds(0, MRCHUNK)
    dg = [pltpu.async_copy(mrc_hbm.at[gi0.at[ch0]], mr_v.at[ch0], sem_g),
          pltpu.async_copy(mrc_hbm.at[gi1.at[ch0]], mc_v.at[ch0], sem_g)]
    for i in range(1, CAP // MRCHUNK):
        ch = pl.ds(i * MRCHUNK, MRCHUNK)

        def _fetch(ch=ch):
            pltpu.sync_copy(mrc_hbm.at[gi0.at[ch]], mr_v.at[ch])
            pltpu.sync_copy(mrc_hbm.at[gi1.at[ch]], mc_v.at[ch])
        pl.when(ngroups > i * 7)(_fetch)
    for d in dbuf:
        d.wait()
    for d in dg:
        d.wait()

    def _group(g, _):
        off = pl.ds(g * 16, 16)
        rv = rs_v[off]
        cv = cs_v[off]
        wv = wn_v[off]
        mr = mr_v[off]
        mc = mc_v[off]
        gvalid = (lane + jnp.broadcast_to(g * 16, (16,))) < cnt

        rbase, rowok, ybandv, ybok = [], [], [], []
        cws, colok = [], []
        for i in range(K):
            wr = rv + (i - PAD)
            rowok.append((wr >= 0) & (wr < H))
            ri = jnp.clip(wr - vstartv, 0, NROWS - 1)
            rbase.append(ri * W)
            yb = wr - lov
            ybok.append((yb >= 0) & (yb < BROWS))
            ybandv.append(yb * W)
        for j in range(K):
            wc = cv + (j - PAD)
            colok.append(wc >= 0)
            cws.append(jnp.maximum(wc, 0))

        esum = zf
        accd = zf
        for k in range(K * K):
            i, j = divmod(k, K)
            gidx = rbase[i] + cws[j]
            ok = rowok[i] & colok[j]
            a = plsc.load_gather(buf_r, [gidx])
            bb = plsc.load_gather(buf_c, [gidx])
            d = plsc.load_gather(buf_d, [gidx])
            a = jnp.where(ok, a, 0.0)
            bb = jnp.where(ok, bb, 0.0)
            d = jnp.where(ok, d, 0.0)
            dr = a - mr
            dc = bb - mc
            e = jnp.exp((dr * dr + dc * dc) * (-1.0 / (TEMP * TEMP)))
            esum = esum + e
            accd = accd + e * d
            e_buf[pl.ds(k * 16, 16)] = e

        inv = 1.0 / (esum + EPS)
        dsoft = accd * inv

        ybc = rv - lov
        cm = gvalid & (ybc >= 0) & (ybc < BROWS)
        plsc.addupdate_scatter(acc_dep, [ybc * W + cv], dsoft, mask=cm)

        wm = gvalid & (wv > 0.5)
        for k in range(K * K):
            i, j = divmod(k, K)
            m = wm & ybok[i] & colok[j]
            sidx = ybandv[i] + jnp.maximum(cv + (j - PAD), 0)
            en = e_buf[pl.ds(k * 16, 16)] * inv
            plsc.addupdate_scatter(acc_num, [sidx], en, mask=m)
            plsc.addupdate_scatter(acc_cnt, [sidx], onef, mask=m)
        return 0
    lax.fori_loop(0, ngroups, _group, 0)

    def _div(i, _):
        for u in range(8):
            off = pl.ds(i * 128 + u * 16, 16)
            acc_num[off] = acc_num[off] / (acc_cnt[off] + EPS)
        return 0
    lax.fori_loop(0, ACC_N // 128, _div, 0)

    do0 = pltpu.async_copy(
        acc_num, out_hbm.at[pl.ds(((b * 2) * H + lo) * W, ACC_N)], sem_o)
    do1 = pltpu.async_copy(
        acc_dep, out_hbm.at[pl.ds(((b * 2 + 1) * H + lo) * W, ACC_N)],
        sem_o)
    do0.wait()
    do1.wait()


_sc_call = pl.kernel(
    _sc_body,
    out_type=jax.ShapeDtypeStruct((B * 2 * H * W,), jnp.float32),
    mesh=plsc.VectorSubcoreMesh(core_axis_name="c", subcore_axis_name="s"),
    compiler_params=pltpu.CompilerParams(needs_layout_passes=False),
    scratch_types=[
        pltpu.VMEM((NROWS * W,), jnp.float32),
        pltpu.VMEM((NROWS * W,), jnp.float32),
        pltpu.VMEM((NROWS * W,), jnp.float32),
        pltpu.VMEM((ACC_N,), jnp.float32),
        pltpu.VMEM((ACC_N,), jnp.float32),
        pltpu.VMEM((ACC_N,), jnp.float32),
        pltpu.VMEM((CAP,), jnp.int32),
        pltpu.VMEM((CAP,), jnp.int32),
        pltpu.VMEM((CAP,), jnp.float32),
        pltpu.VMEM((CAP,), jnp.int32),
        pltpu.VMEM((CAP,), jnp.int32),
        pltpu.VMEM((CAP,), jnp.float32),
        pltpu.VMEM((CAP,), jnp.float32),
        pltpu.VMEM((K * K * 16,), jnp.float32),
        pltpu.VMEM((SP,), jnp.int32),
        pltpu.VMEM((SP,), jnp.int32),
        pltpu.VMEM((SP,), jnp.float32),
        pltpu.SemaphoreType.DMA,
        pltpu.SemaphoreType.DMA,
        pltpu.SemaphoreType.DMA,
        pltpu.SemaphoreType.DMA,
    ],
)


@jax.jit
def kernel(image_depth, rc_tensor, mask_binary, mask_rc_filled,
           batch_row_col_list):
    del mask_binary
    r = batch_row_col_list[:, 0, :].astype(jnp.int32)
    c = batch_row_col_list[:, 1, :].astype(jnp.int32)
    flat = r * W + c
    s = jnp.arange(SP, dtype=jnp.int32)
    dup_later = ((flat[:, :, None] == flat[:, None, :])
                 & (s[None, :, None] < s[None, None, :]))
    winner = (~jnp.any(dup_later, axis=-1)).astype(jnp.float32)
    out = _sc_call(
        image_depth.reshape(B * H * W),
        rc_tensor.reshape(B * 2 * H * W),
        mask_rc_filled.reshape(B * 2 * H * W),
        r.reshape(B * SP), c.reshape(B * SP), winner.reshape(B * SP),
    )
    return out.reshape(B, 2, H, W)


# speedup vs baseline: 1.1592x; 1.0088x over previous
import jax
import jax.numpy as jnp
from jax import lax
from jax.experimental import pallas as pl
from jax.experimental.pallas import tpu as pltpu
from jax.experimental.pallas import tpu_sc as plsc

B, H, W, K, PAD, SP = 2, 384, 512, 7, 3, 768
TEMP, EPS = 10.0, 1e-5

NBAND = 16
BROWS = H // NBAND
NROWS = BROWS + 12
VST_MAX = H - NROWS
NGRP = SP // 16
CAP = SP + 16
MRCHUNK = 112
ACC_N = BROWS * W


def _sc_body(depth_hbm, rc_hbm, mrc_hbm, r_hbm, c_hbm, w_hbm, out_hbm,
             buf_r, buf_c, buf_d, acc_num, acc_cnt, acc_dep,
             rs_v, cs_v, wn_v, gi0, gi1, mr_v, mc_v, e_buf, samp_r,
             samp_c, samp_w, sem_s, sem_b, sem_g, sem_o):
    ci = lax.axis_index("c")
    si = lax.axis_index("s")
    b = ci
    lo = si * BROWS
    vstart = jnp.clip(lo - 6, 0, VST_MAX)

    zf = jnp.zeros((16,), jnp.float32)
    onef = jnp.ones((16,), jnp.float32)
    lane = lax.iota(jnp.int32, 16)
    vstartv = jnp.broadcast_to(vstart, (16,))
    lov = jnp.broadcast_to(lo, (16,))

    dbuf = [
        pltpu.async_copy(
            rc_hbm.at[pl.ds((b * 2 * H + vstart) * W, NROWS * W)],
            buf_r, sem_b),
        pltpu.async_copy(
            rc_hbm.at[pl.ds(((b * 2 + 1) * H + vstart) * W, NROWS * W)],
            buf_c, sem_b),
        pltpu.async_copy(
            depth_hbm.at[pl.ds((b * H + vstart) * W, NROWS * W)],
            buf_d, sem_b),
    ]
    dsamp = [
        pltpu.async_copy(r_hbm.at[pl.ds(b * SP, SP)], samp_r, sem_s),
        pltpu.async_copy(c_hbm.at[pl.ds(b * SP, SP)], samp_c, sem_s),
        pltpu.async_copy(w_hbm.at[pl.ds(b * SP, SP)], samp_w, sem_s),
    ]

    def _acc_zero(i, _):
        for u in range(8):
            off = pl.ds(i * 128 + u * 16, 16)
            acc_num[off] = zf
            acc_cnt[off] = zf
            acc_dep[off] = zf
        return 0
    lax.fori_loop(0, ACC_N // 128, _acc_zero, 0)

    sent_r = jnp.broadcast_to(vstart, (16,))
    zi = jnp.zeros((16,), jnp.int32)

    def _fill(k, _):
        rs_v[pl.ds(k * 16, 16)] = sent_r
        cs_v[pl.ds(k * 16, 16)] = zi
        wn_v[pl.ds(k * 16, 16)] = zf
        return 0
    lax.fori_loop(0, CAP // 16, _fill, 0)

    for d in dsamp:
        d.wait()

    def _compact(g, cnt):
        off = pl.ds(g * 16, 16)
        rv = samp_r[off]
        cv = samp_c[off]
        wv = samp_w[off]
        sel = (rv >= lov - 3) & (rv < lov + (BROWS + 3))
        pos = plsc.cumsum(sel.astype(jnp.int32))
        dest = cnt + pos - 1
        plsc.store_scatter(rs_v, [dest], rv, mask=sel)
        plsc.store_scatter(cs_v, [dest], cv, mask=sel)
        plsc.store_scatter(wn_v, [dest], wv, mask=sel)
        return cnt + plsc.all_reduce_population_count(sel)
    cnt = lax.fori_loop(0, NGRP, _compact, jnp.zeros((16,), jnp.int32))
    ngroups = ((cnt + 15) // 16)[0]

    basev = jnp.broadcast_to(b * (2 * H * W), (16,))

    def _gidx(k, _):
        off = pl.ds(k * 16, 16)
        fi = rs_v[off] * W + cs_v[off]
        gi0[off] = basev + fi
        gi1[off] = basev + (H * W) + fi
        return 0
    lax.fori_loop(0, CAP // 16, _gidx, 0)
    ch0 = pl.ds(0, MRCHUNK)
    dg = [pltpu.async_copy(mrc_hbm.at[gi0.at[ch0]], mr_v.at[ch0], sem_g),
          pltpu.async_copy(mrc_hbm.at[gi1.at[ch0]], mc_v.at[ch0], sem_g)]
    for i in range(1, CAP // MRCHUNK):
        ch = pl.ds(i * MRCHUNK, MRCHUNK)

        def _fetch(ch=ch):
            pltpu.sync_copy(mrc_hbm.at[gi0.at[ch]], mr_v.at[ch])
            pltpu.sync_copy(mrc_hbm.at[gi1.at[ch]], mc_v.at[ch])
        pl.when(ngroups > i * 7)(_fetch)
    for d in dbuf:
        d.wait()
    for d in dg:
        d.wait()

    def _group(g, _):
        off = pl.ds(g * 16, 16)
        rv = rs_v[off]
        cv = cs_v[off]
        wv = wn_v[off]
        mr = mr_v[off]
        mc = mc_v[off]
        gvalid = (lane + jnp.broadcast_to(g * 16, (16,))) < cnt

        rbase, rowok, ybandv, ybok = [], [], [], []
        cws, colok = [], []
        for i in range(K):
            wr = rv + (i - PAD)
            rowok.append((wr >= 0) & (wr < H))
            ri = jnp.clip(wr - vstartv, 0, NROWS - 1)
            rbase.append(ri * W)
            yb = wr - lov
            ybok.append((yb >= 0) & (yb < BROWS))
            ybandv.append(yb * W)
        for j in range(K):
            wc = cv + (j - PAD)
            colok.append(wc >= 0)
            cws.append(jnp.maximum(wc, 0))

        esum = zf
        accd = zf
        for k in range(K * K):
            i, j = divmod(k, K)
            gidx = rbase[i] + cws[j]
            ok = rowok[i] & colok[j]
            a = plsc.load_gather(buf_r, [gidx])
            bb = plsc.load_gather(buf_c, [gidx])
            d = plsc.load_gather(buf_d, [gidx])
            a = jnp.where(ok, a, 0.0)
            bb = jnp.where(ok, bb, 0.0)
            d = jnp.where(ok, d, 0.0)
            dr = a - mr
            dc = bb - mc
            e = jnp.exp((dr * dr + dc * dc) * (-1.0 / (TEMP * TEMP)))
            esum = esum + e
            accd = accd + e * d
            e_buf[pl.ds(k * 16, 16)] = e

        inv = 1.0 / (esum + EPS)
        dsoft = accd * inv

        ybc = rv - lov
        cm = gvalid & (ybc >= 0) & (ybc < BROWS)
        plsc.addupdate_scatter(acc_dep, [ybc * W + cv], dsoft, mask=cm)

        wm = gvalid & (wv > 0.5)
        for k in range(K * K):
            i, j = divmod(k, K)
            m = wm & ybok[i] & colok[j]
            sidx = ybandv[i] + jnp.maximum(cv + (j - PAD), 0)
            en = e_buf[pl.ds(k * 16, 16)] * inv
            plsc.addupdate_scatter(acc_num, [sidx], en, mask=m)
            plsc.addupdate_scatter(acc_cnt, [sidx], onef, mask=m)
        return 0
    lax.fori_loop(0, ngroups, _group, 0)

    do1 = pltpu.async_copy(
        acc_dep, out_hbm.at[pl.ds(((b * 2 + 1) * H + lo) * W, ACC_N)],
        sem_o)

    def _div(i, _):
        for u in range(8):
            off = pl.ds(i * 128 + u * 16, 16)
            acc_num[off] = acc_num[off] / (acc_cnt[off] + EPS)
        return 0
    lax.fori_loop(0, ACC_N // 128, _div, 0)

    do0 = pltpu.async_copy(
        acc_num, out_hbm.at[pl.ds(((b * 2) * H + lo) * W, ACC_N)], sem_o)
    do1.wait()
    do0.wait()


_sc_call = pl.kernel(
    _sc_body,
    out_type=jax.ShapeDtypeStruct((B * 2 * H * W,), jnp.float32),
    mesh=plsc.VectorSubcoreMesh(core_axis_name="c", subcore_axis_name="s"),
    compiler_params=pltpu.CompilerParams(needs_layout_passes=False),
    scratch_types=[
        pltpu.VMEM((NROWS * W,), jnp.float32),
        pltpu.VMEM((NROWS * W,), jnp.float32),
        pltpu.VMEM((NROWS * W,), jnp.float32),
        pltpu.VMEM((ACC_N,), jnp.float32),
        pltpu.VMEM((ACC_N,), jnp.float32),
        pltpu.VMEM((ACC_N,), jnp.float32),
        pltpu.VMEM((CAP,), jnp.int32),
        pltpu.VMEM((CAP,), jnp.int32),
        pltpu.VMEM((CAP,), jnp.float32),
        pltpu.VMEM((CAP,), jnp.int32),
        pltpu.VMEM((CAP,), jnp.int32),
        pltpu.VMEM((CAP,), jnp.float32),
        pltpu.VMEM((CAP,), jnp.float32),
        pltpu.VMEM((K * K * 16,), jnp.float32),
        pltpu.VMEM((SP,), jnp.int32),
        pltpu.VMEM((SP,), jnp.int32),
        pltpu.VMEM((SP,), jnp.float32),
        pltpu.SemaphoreType.DMA,
        pltpu.SemaphoreType.DMA,
        pltpu.SemaphoreType.DMA,
        pltpu.SemaphoreType.DMA,
    ],
)


@jax.jit
def kernel(image_depth, rc_tensor, mask_binary, mask_rc_filled,
           batch_row_col_list):
    del mask_binary
    r = batch_row_col_list[:, 0, :].astype(jnp.int32)
    c = batch_row_col_list[:, 1, :].astype(jnp.int32)
    flat = r * W + c
    s = jnp.arange(SP, dtype=jnp.int32)
    dup_later = ((flat[:, :, None] == flat[:, None, :])
                 & (s[None, :, None] < s[None, None, :]))
    winner = (~jnp.any(dup_later, axis=-1)).astype(jnp.float32)
    out = _sc_call(
        image_depth.reshape(B * H * W),
        rc_tensor.reshape(B * 2 * H * W),
        mask_rc_filled.reshape(B * 2 * H * W),
        r.reshape(B * SP), c.reshape(B * SP), winner.reshape(B * SP),
    )
    return out.reshape(B, 2, H, W)
